# Initial kernel scaffold; baseline (speedup 1.0000x reference)
import jax
import jax.numpy as jnp
from jax.experimental import pallas as pl


def kernel(x, gate_w, w1, w2, w3, sw1, sw2, sw3):
    raise NotImplementedError("write your pallas kernel here")


# speedup vs baseline: 1.0821x; 1.0821x over previous
import jax
import jax.numpy as jnp
from jax.experimental import pallas as pl

T = 2048
DIM = 1024
HID = 1024
E = 8

BT = 1024
BT2 = 512

_HIGH = jax.lax.Precision.HIGHEST


def _router_kernel(x_ref, gw_ref, comb_ref):
    x = x_ref[...].astype(jnp.bfloat16)
    gw = gw_ref[...].astype(jnp.bfloat16)
    logits = jax.lax.dot_general(x, gw, (((1,), (1,)), ((), ())),
                                 preferred_element_type=jnp.float32)
    scores = jax.nn.sigmoid(logits)
    lane = jax.lax.broadcasted_iota(jnp.int32, scores.shape, 1)
    m1 = jnp.max(scores, axis=1, keepdims=True)
    i1 = jnp.min(jnp.where(scores >= m1, lane, E), axis=1, keepdims=True)
    sel1 = lane == i1
    masked = jnp.where(sel1, -1.0, scores)
    m2 = jnp.max(masked, axis=1, keepdims=True)
    i2 = jnp.min(jnp.where(masked >= m2, lane, E), axis=1, keepdims=True)
    sel2 = lane == i2
    denom = m1 + m2 + 1e-20
    comb_ref[...] = (jnp.where(sel1, m1 / denom, 0.0)
                     + jnp.where(sel2, m2 / denom, 0.0))


def _moe_kernel(xb_ref, comb_ref, w1_ref, w3_ref, w2_ref, out_ref):
    e = pl.program_id(1)
    x = xb_ref[...]
    h1 = jax.lax.dot_general(x, w1_ref[0], (((1,), (1,)), ((), ())),
                             preferred_element_type=jnp.float32)
    h3 = jax.lax.dot_general(x, w3_ref[0], (((1,), (1,)), ((), ())),
                             preferred_element_type=jnp.float32)
    h = (h1 * jax.nn.sigmoid(h1) * h3).astype(jnp.bfloat16)
    y = jax.lax.dot_general(h, w2_ref[0], (((1,), (1,)), ((), ())),
                            preferred_element_type=jnp.float32)
    comb = comb_ref[...]
    lane = jax.lax.broadcasted_iota(jnp.int32, comb.shape, 1)
    w = jnp.sum(jnp.where(lane == e, comb, 0.0), axis=1, keepdims=True)

    @pl.when(e == 0)
    def _():
        out_ref[...] = jnp.zeros_like(out_ref)

    out_ref[...] += w * y


def _shared_kernel(xb_ref, routed_ref, sw1_ref, sw3_ref, sw2_ref, out_ref):
    x = xb_ref[...]
    h1 = jax.lax.dot_general(x, sw1_ref[...], (((1,), (1,)), ((), ())),
                             preferred_element_type=jnp.float32)
    h3 = jax.lax.dot_general(x, sw3_ref[...], (((1,), (1,)), ((), ())),
                             preferred_element_type=jnp.float32)
    h = (h1 * jax.nn.sigmoid(h1) * h3).astype(jnp.bfloat16)
    y = jax.lax.dot_general(h, sw2_ref[...], (((1,), (1,)), ((), ())),
                            preferred_element_type=jnp.float32)
    out_ref[...] = routed_ref[...] + y


def _run(x, gate_w, w1, w2, w3, sw1, sw2, sw3, interpret=False):
    bf16 = jnp.bfloat16
    xb = x.astype(bf16)
    comb = pl.pallas_call(
        _router_kernel,
        out_shape=jax.ShapeDtypeStruct((T, E), jnp.float32),
        interpret=interpret,
    )(x, gate_w)
    routed = pl.pallas_call(
        _moe_kernel,
        grid=(T // BT, E),
        in_specs=[
            pl.BlockSpec((BT, DIM), lambda i, e: (i, 0)),
            pl.BlockSpec((BT, E), lambda i, e: (i, 0)),
            pl.BlockSpec((1, HID, DIM), lambda i, e: (e, 0, 0)),
            pl.BlockSpec((1, HID, DIM), lambda i, e: (e, 0, 0)),
            pl.BlockSpec((1, DIM, HID), lambda i, e: (e, 0, 0)),
        ],
        out_specs=pl.BlockSpec((BT, DIM), lambda i, e: (i, 0)),
        out_shape=jax.ShapeDtypeStruct((T, DIM), jnp.float32),
        interpret=interpret,
    )(xb, comb, w1.astype(bf16), w3.astype(bf16), w2.astype(bf16))
    out = pl.pallas_call(
        _shared_kernel,
        grid=(T // BT2,),
        in_specs=[
            pl.BlockSpec((BT2, DIM), lambda i: (i, 0)),
            pl.BlockSpec((BT2, DIM), lambda i: (i, 0)),
            pl.BlockSpec((HID, DIM), lambda i: (0, 0)),
            pl.BlockSpec((HID, DIM), lambda i: (0, 0)),
            pl.BlockSpec((DIM, HID), lambda i: (0, 0)),
        ],
        out_specs=pl.BlockSpec((BT2, DIM), lambda i: (i, 0)),
        out_shape=jax.ShapeDtypeStruct((T, DIM), jnp.float32),
        interpret=interpret,
    )(xb, routed, sw1.astype(bf16), sw3.astype(bf16), sw2.astype(bf16))
    return out


def kernel(x, gate_w, w1, w2, w3, sw1, sw2, sw3):
    return _run(x, gate_w, w1, w2, w3, sw1, sw2, sw3)
